# Initial kernel scaffold; baseline (speedup 1.0000x reference)
import jax
import jax.numpy as jnp
from jax.experimental import pallas as pl


def kernel(features, edge_weights, threashold, Wl1, bl1, Wr1, br1, We1, att1, b1, Wl2, bl2, Wr2, br2, We2, att2, b2, Wl3, bl3, Wr3, br3, We3, att3, b3, Wl4, bl4, Wr4, br4, We4, att4, b4, cw1, cb1, cw2, cb2, cw3, cb3, lw, lb):
    raise NotImplementedError("write your pallas kernel here")


# speedup vs baseline: 358.1428x; 358.1428x over previous
import functools

import jax
import jax.numpy as jnp
from jax.experimental import pallas as pl
from jax.experimental.pallas import tpu as pltpu

_N = 2048
_BJ = 256


def _wmean_body(ew_ref, cutoff_ref, out_ref):
    ew = ew_ref[...]
    mask = ew > cutoff_ref[0, 0]
    s = jnp.sum(jnp.where(mask, ew, 0.0))
    cnt = jnp.sum(mask.astype(jnp.float32))
    out_ref[...] = jnp.broadcast_to(s / cnt, (1, 1))


def _wmean(ew, cutoff):
    return pl.pallas_call(
        _wmean_body,
        in_specs=[
            pl.BlockSpec((_N, _N), lambda: (0, 0)),
            pl.BlockSpec(memory_space=pltpu.SMEM),
        ],
        out_specs=pl.BlockSpec((1, 1), lambda: (0, 0)),
        out_shape=jax.ShapeDtypeStruct((1, 1), jnp.float32),
    )(ew, cutoff)


def _prep_body(x_ref, xT_ref, Wl_ref, WlT_ref, Wr_ref, bl_row_ref, bl_col_ref,
               br_row_ref, xl_ref, xlT_ref, xr_ref, *, norm):
    x = x_ref[...]
    xT = xT_ref[...]
    if norm:
        n = x.shape[0]
        mu = jnp.mean(x, axis=0, keepdims=True)
        sd = jnp.sqrt(jnp.sum((x - mu) ** 2, axis=0, keepdims=True) / (n - 1))
        x = (x - mu) / sd + 1.0
        muT = jnp.mean(xT, axis=1, keepdims=True)
        sdT = jnp.sqrt(jnp.sum((xT - muT) ** 2, axis=1, keepdims=True) / (n - 1))
        xT = (xT - muT) / sdT + 1.0
    xl_ref[...] = jnp.dot(x, Wl_ref[...], preferred_element_type=jnp.float32) + bl_row_ref[...]
    xlT_ref[...] = jnp.dot(WlT_ref[...], xT, preferred_element_type=jnp.float32) + bl_col_ref[...]
    xr_ref[...] = jnp.dot(x, Wr_ref[...], preferred_element_type=jnp.float32) + br_row_ref[...]


def _prep(x, xT, Wl, bl, Wr, br, norm):
    din, dout = Wl.shape
    body = functools.partial(_prep_body, norm=norm)
    return pl.pallas_call(
        body,
        in_specs=[
            pl.BlockSpec((_N, din), lambda: (0, 0)),
            pl.BlockSpec((din, _N), lambda: (0, 0)),
            pl.BlockSpec((din, dout), lambda: (0, 0)),
            pl.BlockSpec((dout, din), lambda: (0, 0)),
            pl.BlockSpec((din, dout), lambda: (0, 0)),
            pl.BlockSpec((1, dout), lambda: (0, 0)),
            pl.BlockSpec((dout, 1), lambda: (0, 0)),
            pl.BlockSpec((1, dout), lambda: (0, 0)),
        ],
        out_specs=[
            pl.BlockSpec((_N, dout), lambda: (0, 0)),
            pl.BlockSpec((dout, _N), lambda: (0, 0)),
            pl.BlockSpec((_N, dout), lambda: (0, 0)),
        ],
        out_shape=[
            jax.ShapeDtypeStruct((_N, dout), jnp.float32),
            jax.ShapeDtypeStruct((dout, _N), jnp.float32),
            jax.ShapeDtypeStruct((_N, dout), jnp.float32),
        ],
    )(x, xT, Wl, Wl.T, Wr, bl.reshape(1, dout), bl.reshape(dout, 1),
      br.reshape(1, dout))


def _attn_body(xl_ref, xlT_ref, xlb_ref, xr_ref, ewT_ref, We_v_ref,
               att_row_ref, att_col_ref, bias_ref, We_s_ref, att_s_ref,
               cutoff_ref, wmean_ref, out_ref, *, dout):
    xlT = xlT_ref[...]
    ew = ewT_ref[...]
    xr_b = xr_ref[...]
    att_row = att_row_ref[...]
    att_col = att_col_ref[...]

    A_row = jnp.dot(att_row, xlT, preferred_element_type=jnp.float32)
    B_col = jnp.dot(xr_b, att_col, preferred_element_type=jnp.float32)
    s = jnp.sum(We_v_ref[...] * att_row)
    base = A_row + B_col + ew * s

    acc = jnp.zeros(ew.shape, jnp.float32)
    for c in range(dout):
        acc = acc + jnp.abs(xlT[c:c + 1, :] + xr_b[:, c:c + 1]
                            + ew * We_s_ref[0, c]) * att_s_ref[0, c]
    LT = 0.6 * base + 0.4 * acc
    LT = jnp.where(ew > cutoff_ref[0, 0], LT, -1e30)

    xl_b = xlb_ref[...]
    zs = xl_b + xr_b + wmean_ref[0, 0] * We_v_ref[...]
    zs = jnp.where(zs >= 0, zs, 0.2 * zs)
    ls = jnp.dot(zs, att_col, preferred_element_type=jnp.float32)

    m = jnp.maximum(jnp.max(LT, axis=1, keepdims=True), ls)
    ex = jnp.exp(LT - m)
    exs = jnp.exp(ls - m)
    denom = jnp.sum(ex, axis=1, keepdims=True) + exs
    aggr = jnp.dot(ex, xl_ref[...], preferred_element_type=jnp.float32)
    out_ref[...] = (aggr + exs * xl_b) / denom + bias_ref[...]


def _attn(xl, xlT, xr, ewT, We, att, bias, cutoff, wmean):
    dout = xl.shape[1]
    body = functools.partial(_attn_body, dout=dout)
    att_row = att.reshape(1, dout)
    return pl.pallas_call(
        body,
        grid=(_N // _BJ,),
        in_specs=[
            pl.BlockSpec((_N, dout), lambda j: (0, 0)),
            pl.BlockSpec((dout, _N), lambda j: (0, 0)),
            pl.BlockSpec((_BJ, dout), lambda j: (j, 0)),
            pl.BlockSpec((_BJ, dout), lambda j: (j, 0)),
            pl.BlockSpec((_BJ, _N), lambda j: (j, 0)),
            pl.BlockSpec((1, dout), lambda j: (0, 0)),
            pl.BlockSpec((1, dout), lambda j: (0, 0)),
            pl.BlockSpec((dout, 1), lambda j: (0, 0)),
            pl.BlockSpec((1, dout), lambda j: (0, 0)),
            pl.BlockSpec(memory_space=pltpu.SMEM),
            pl.BlockSpec(memory_space=pltpu.SMEM),
            pl.BlockSpec(memory_space=pltpu.SMEM),
            pl.BlockSpec(memory_space=pltpu.SMEM),
        ],
        out_specs=pl.BlockSpec((_BJ, dout), lambda j: (j, 0)),
        out_shape=jax.ShapeDtypeStruct((_N, dout), jnp.float32),
    )(xl, xlT, xl, xr, ewT, We, att_row, att.reshape(dout, 1),
      bias.reshape(1, dout), We, att_row, cutoff, wmean)


def _head_body(x4_ref, cw1_ref, cw2_ref, cw3_ref, cb_ref, lw_row_ref, out_ref):
    x5 = jnp.mean(x4_ref[...], axis=0, keepdims=True)
    y1 = jnp.zeros((1, 101), jnp.float32)
    for k in range(10):
        y1 = y1 + x5[:, 3 * k:3 * k + 101] * cw1_ref[0, k]
    y1 = jnp.maximum(y1 + cb_ref[0, 0], 0.0)
    y2 = jnp.zeros((1, 74), jnp.float32)
    for k in range(10):
        y2 = y2 + y1[:, 3 * k:3 * k + 74] * cw2_ref[0, k]
    y2 = jnp.maximum(y2 + cb_ref[0, 1], 0.0)
    y3 = jnp.zeros((1, 47), jnp.float32)
    for k in range(10):
        y3 = y3 + y2[:, 3 * k:3 * k + 47] * cw3_ref[0, k]
    y3 = jnp.maximum(y3 + cb_ref[0, 2], 0.0)
    out_ref[...] = (jnp.sum(y3 * lw_row_ref[...], axis=1, keepdims=True)
                    + cb_ref[0, 3])


def _head(x4, cw1, cb1, cw2, cb2, cw3, cb3, lw, lb):
    lw_row = jnp.zeros((1, 47), jnp.float32).at[0, ::2].set(lw[:, 0])
    cb = jnp.stack([cb1[0], cb2[0], cb3[0], lb[0]]).reshape(1, 4)
    return pl.pallas_call(
        _head_body,
        in_specs=[
            pl.BlockSpec((_N, 128), lambda: (0, 0)),
            pl.BlockSpec(memory_space=pltpu.SMEM),
            pl.BlockSpec(memory_space=pltpu.SMEM),
            pl.BlockSpec(memory_space=pltpu.SMEM),
            pl.BlockSpec(memory_space=pltpu.SMEM),
            pl.BlockSpec((1, 47), lambda: (0, 0)),
        ],
        out_specs=pl.BlockSpec((1, 1), lambda: (0, 0)),
        out_shape=jax.ShapeDtypeStruct((1, 1), jnp.float32),
    )(x4, cw1.reshape(1, 10), cw2.reshape(1, 10), cw3.reshape(1, 10), cb,
      lw_row)


def kernel(features, edge_weights, threashold, Wl1, bl1, Wr1, br1, We1, att1,
           b1, Wl2, bl2, Wr2, br2, We2, att2, b2, Wl3, bl3, Wr3, br3, We3,
           att3, b3, Wl4, bl4, Wr4, br4, We4, att4, b4, cw1, cb1, cw2, cb2,
           cw3, cb3, lw, lb):
    cutoff = (1.0 / jnp.asarray(threashold).astype(jnp.float32)).reshape(1, 1)
    ewT = edge_weights.T
    wmean = _wmean(edge_weights, cutoff)

    layers = [
        (Wl1, bl1, Wr1, br1, We1, att1, b1),
        (Wl2, bl2, Wr2, br2, We2, att2, b2),
        (Wl3, bl3, Wr3, br3, We3, att3, b3),
        (Wl4, bl4, Wr4, br4, We4, att4, b4),
    ]
    x = features
    xT = features.T
    for i, (Wl, bl, Wr, br, We, att, bias) in enumerate(layers):
        xl, xlT, xr = _prep(x, xT, Wl, bl, Wr, br, norm=(i > 0))
        x = _attn(xl, xlT, xr, ewT, We, att, bias, cutoff, wmean)
        if i < 3:
            xT = x.T
    return _head(x, cw1, cb1, cw2, cb2, cw3, cb3, lw, lb)
